# baseline (speedup 1.0000x reference)
import functools

import jax
import jax.numpy as jnp
from jax import lax
from jax.experimental import pallas as pl
from jax.experimental.pallas import tpu as pltpu
from jax.experimental.pallas import tpu_sc as plsc

NUM_PROTO = 8192
DIM = 256
TOPK = 8
B = 4096

BR = 128
MASKED = -3.0

NSEG = 8
SEG = NUM_PROTO // NSEG

SORT8 = [(0, 1), (2, 3), (4, 5), (6, 7), (0, 2), (1, 3), (4, 6), (5, 7),
         (1, 2), (5, 6), (0, 4), (3, 7), (1, 5), (2, 6), (1, 4), (3, 6),
         (2, 4), (3, 5), (3, 4)]

SPLIT = 2
BH = B // SPLIT
NW = 32
ROWS_PER_W = BH // NW
CH = 16
NCHUNK = ROWS_PER_W // CH
NSUPER = NCHUNK // 2
NL = 16


def _topk_tournament(logits):
    s = [logits[:, m * SEG:(m + 1) * SEG] for m in range(NSEG)]
    col0 = lax.broadcasted_iota(jnp.int32, (BR, SEG), 1)
    si = [col0 + m * SEG for m in range(NSEG)]
    for a, b in SORT8:
        swap = s[a] < s[b]
        hi = jnp.maximum(s[a], s[b])
        lo = jnp.minimum(s[a], s[b])
        ia = jnp.where(swap, si[b], si[a])
        ib = jnp.where(swap, si[a], si[b])
        s[a], s[b] = hi, lo
        si[a], si[b] = ia, ib
    vals, idxs = [], []
    f, fi, rest, resti = s[0], si[0], s[1:], si[1:]
    for it in range(TOPK):
        v = jnp.max(f, axis=1)
        eq = f == v[:, None]
        i = jnp.sum(jnp.where(eq, fi, 0), axis=1)
        vals.append(v)
        idxs.append(i)
        if it < TOPK - 1:
            f = jnp.where(eq, rest[0], f)
            fi = jnp.where(eq, resti[0], fi)
            for j in range(len(rest) - 1):
                rest[j] = jnp.where(eq, rest[j + 1], rest[j])
                resti[j] = jnp.where(eq, resti[j + 1], resti[j])
            rest[-1] = jnp.where(eq, jnp.full_like(rest[-1], MASKED), rest[-1])
    return jnp.stack(vals, 1), jnp.stack(idxs, 1)


def _tc_kernel(fnb_ref, cbn_ref, assign_ref, logits_ref, idx_ref, wx_ref):
    logits = jax.lax.dot_general(
        fnb_ref[...], cbn_ref[...],
        (((1,), (1,)), ((), ())),
        preferred_element_type=jnp.float32,
    )
    logits_ref[...] = logits

    vstack, istack = _topk_tournament(logits)
    m = vstack[:, 0]
    p = jnp.exp(vstack - m[:, None])
    z = jnp.sum(p, axis=1)
    idx_ref[...] = istack

    wnorm = p / z[:, None]
    sel = (lax.broadcasted_iota(jnp.int32, (TOPK, TOPK * NL), 1) // NL
           == lax.broadcasted_iota(jnp.int32, (TOPK, TOPK * NL), 0))
    wx_ref[...] = jax.lax.dot_general(
        wnorm, sel.astype(jnp.float32), (((1,), (0,)), ((), ())),
        preferred_element_type=jnp.float32,
        precision=jax.lax.Precision.HIGHEST,
    )

    selected = logits >= vstack[:, TOPK - 1][:, None]
    assign_ref[...] = jnp.where(
        selected, jnp.exp(logits - m[:, None]) / z[:, None], 0.0)


def _sc_proto(cb_hbm, idx_hbm, wx_hbm, out_hbm,
              idx_a, idx_b, wx_a, wx_b, rows_a, rows_b, acc_v,
              sem_a, sem_b):
    wid = lax.axis_index("s") * 2 + lax.axis_index("c")
    base = wid * ROWS_PER_W

    def fetch(c, idx_v, wx_v, rows_v, sem):
        rb = base + c * CH
        pltpu.sync_copy(idx_hbm.at[pl.ds(rb * TOPK, CH * TOPK)], idx_v)
        pltpu.sync_copy(wx_hbm.at[pl.ds(rb * TOPK, CH * TOPK)], wx_v)
        return pltpu.async_copy(cb_hbm.at[idx_v], rows_v, sem)

    def compute(c, wx_v, rows_v):
        rb = base + c * CH

        def row_body(r, _):
            accs = [jnp.zeros((NL,), jnp.float32) for _ in range(DIM // NL)]
            for k in range(TOPK):
                wv = wx_v[r * TOPK + k]
                for d in range(DIM // NL):
                    accs[d] = accs[d] + wv * rows_v[r * TOPK + k,
                                                    pl.ds(d * NL, NL)]
            for d in range(DIM // NL):
                acc_v[r, pl.ds(d * NL, NL)] = accs[d]
            return ()

        lax.fori_loop(0, CH, row_body, (), unroll=False)
        pltpu.sync_copy(acc_v, out_hbm.at[pl.ds(rb, CH)])

    last = NCHUNK - 1
    fetch(0, idx_a, wx_a, rows_a, sem_a)
    fetch(1, idx_b, wx_b, rows_b, sem_b)

    def superstep(g, _):
        c0 = 2 * g
        pltpu.make_async_copy(cb_hbm.at[idx_a], rows_a, sem_a).wait()
        compute(c0, wx_a, rows_a)
        fetch(jnp.minimum(c0 + 2, last), idx_a, wx_a, rows_a, sem_a)
        pltpu.make_async_copy(cb_hbm.at[idx_b], rows_b, sem_b).wait()
        compute(c0 + 1, wx_b, rows_b)
        fetch(jnp.minimum(c0 + 3, last), idx_b, wx_b, rows_b, sem_b)
        return ()

    lax.fori_loop(0, NSUPER, superstep, (), unroll=False)
    pltpu.make_async_copy(cb_hbm.at[idx_a], rows_a, sem_a).wait()
    pltpu.make_async_copy(cb_hbm.at[idx_b], rows_b, sem_b).wait()


@jax.jit
def kernel(feat_vec, codebook):
    fn = feat_vec / jnp.maximum(
        jnp.linalg.norm(feat_vec, axis=-1, keepdims=True), 1e-12)
    cbn = codebook / jnp.maximum(
        jnp.linalg.norm(codebook, axis=-1, keepdims=True), 1e-12)
    fnb = fn.astype(jnp.bfloat16)
    cbnb = cbn.astype(jnp.bfloat16)

    tc_call = pl.pallas_call(
        _tc_kernel,
        grid=(BH // BR,),
        in_specs=[
            pl.BlockSpec((BR, DIM), lambda i: (i, 0)),
            pl.BlockSpec((NUM_PROTO, DIM), lambda i: (0, 0)),
        ],
        out_specs=[
            pl.BlockSpec((BR, NUM_PROTO), lambda i: (i, 0)),
            pl.BlockSpec((BR, NUM_PROTO), lambda i: (i, 0)),
            pl.BlockSpec((BR, TOPK), lambda i: (i, 0)),
            pl.BlockSpec((BR, TOPK * NL), lambda i: (i, 0)),
        ],
        out_shape=[
            jax.ShapeDtypeStruct((BH, NUM_PROTO), jnp.float32),
            jax.ShapeDtypeStruct((BH, NUM_PROTO), jnp.float32),
            jax.ShapeDtypeStruct((BH, TOPK), jnp.int32),
            jax.ShapeDtypeStruct((BH, TOPK * NL), jnp.float32),
        ],
    )

    mesh = plsc.VectorSubcoreMesh(core_axis_name="c", subcore_axis_name="s")
    sc_call = pl.kernel(
        _sc_proto,
        out_type=jax.ShapeDtypeStruct((BH, DIM), jnp.float32),
        mesh=mesh,
        scratch_types=[
            pltpu.VMEM((CH * TOPK,), jnp.int32),
            pltpu.VMEM((CH * TOPK,), jnp.int32),
            pltpu.VMEM((CH * TOPK, NL), jnp.float32),
            pltpu.VMEM((CH * TOPK, NL), jnp.float32),
            pltpu.VMEM((CH * TOPK, DIM), jnp.float32),
            pltpu.VMEM((CH * TOPK, DIM), jnp.float32),
            pltpu.VMEM((CH, DIM), jnp.float32),
            pltpu.SemaphoreType.DMA,
            pltpu.SemaphoreType.DMA,
        ],
    )

    assign = jnp.zeros((B, NUM_PROTO), jnp.float32)
    logits = jnp.zeros((B, NUM_PROTO), jnp.float32)
    proto = jnp.zeros((B, DIM), jnp.float32)
    for h in range(SPLIT):
        a_h, l_h, idx_h, wx_h = tc_call(fnb[h * BH:(h + 1) * BH], cbnb)
        p_h = sc_call(codebook, idx_h.reshape(BH * TOPK),
                      wx_h.reshape(BH * TOPK, NL))
        assign = lax.dynamic_update_slice(assign, a_h, (h * BH, 0))
        logits = lax.dynamic_update_slice(logits, l_h, (h * BH, 0))
        proto = lax.dynamic_update_slice(proto, p_h, (h * BH, 0))
    return (assign, proto, logits)


# speedup vs baseline: 1.5598x; 1.5598x over previous
import functools

import jax
import jax.numpy as jnp
from jax import lax
from jax.experimental import pallas as pl
from jax.experimental.pallas import tpu as pltpu
from jax.experimental.pallas import tpu_sc as plsc

NUM_PROTO = 8192
DIM = 256
TOPK = 8
B = 4096

BR = 128
MASKED = -3.0

NSEG = 8
SEG = NUM_PROTO // NSEG

SORT8 = [(0, 1), (2, 3), (4, 5), (6, 7), (0, 2), (1, 3), (4, 6), (5, 7),
         (1, 2), (5, 6), (0, 4), (3, 7), (1, 5), (2, 6), (1, 4), (3, 6),
         (2, 4), (3, 5), (3, 4)]

NW = 32
ROWS_PER_W = B // NW
CH = 16
NCHUNK = ROWS_PER_W // CH
NSUPER = NCHUNK // 2
NL = 16


def _topk_tournament(logits):
    s = [logits[:, m * SEG:(m + 1) * SEG] for m in range(NSEG)]
    col0 = lax.broadcasted_iota(jnp.int32, (BR, SEG), 1)
    si = [col0 + m * SEG for m in range(NSEG)]
    for a, b in SORT8:
        swap = s[a] < s[b]
        hi = jnp.maximum(s[a], s[b])
        lo = jnp.minimum(s[a], s[b])
        ia = jnp.where(swap, si[b], si[a])
        ib = jnp.where(swap, si[a], si[b])
        s[a], s[b] = hi, lo
        si[a], si[b] = ia, ib
    vals, idxs = [], []
    f, fi, rest, resti = s[0], si[0], s[1:], si[1:]
    for it in range(TOPK):
        v = jnp.max(f, axis=1)
        eq = f == v[:, None]
        i = jnp.sum(jnp.where(eq, fi, 0), axis=1)
        vals.append(v)
        idxs.append(i)
        if it < TOPK - 1:
            f = jnp.where(eq, rest[0], f)
            fi = jnp.where(eq, resti[0], fi)
            for j in range(len(rest) - 1):
                rest[j] = jnp.where(eq, rest[j + 1], rest[j])
                resti[j] = jnp.where(eq, resti[j + 1], resti[j])
            rest[-1] = jnp.where(eq, jnp.full_like(rest[-1], MASKED), rest[-1])
    return jnp.stack(vals, 1), jnp.stack(idxs, 1)


def _tc_kernel(fnb_ref, cbn_ref, assign_ref, logits_ref, idx_ref, wx_ref):
    logits = jax.lax.dot_general(
        fnb_ref[...], cbn_ref[...],
        (((1,), (1,)), ((), ())),
        preferred_element_type=jnp.float32,
    )
    logits_ref[...] = logits

    vstack, istack = _topk_tournament(logits)
    m = vstack[:, 0]
    p = jnp.exp(vstack - m[:, None])
    z = jnp.sum(p, axis=1)
    idx_ref[...] = istack

    wnorm = p / z[:, None]
    sel = (lax.broadcasted_iota(jnp.int32, (TOPK, TOPK * NL), 1) // NL
           == lax.broadcasted_iota(jnp.int32, (TOPK, TOPK * NL), 0))
    wx_ref[...] = jax.lax.dot_general(
        wnorm, sel.astype(jnp.float32), (((1,), (0,)), ((), ())),
        preferred_element_type=jnp.float32,
        precision=jax.lax.Precision.HIGHEST,
    )

    selected = logits >= vstack[:, TOPK - 1][:, None]
    assign_ref[...] = jnp.where(
        selected, jnp.exp(logits - m[:, None]) / z[:, None], 0.0)


def _sc_proto(cb_hbm, idx_hbm, wx_hbm, out_hbm,
              idx_a, idx_b, wx_a, wx_b, rows_a, rows_b, acc_v,
              sem_a, sem_b):
    wid = lax.axis_index("s") * 2 + lax.axis_index("c")
    base = wid * ROWS_PER_W

    def fetch(c, idx_v, wx_v, rows_v, sem):
        rb = base + c * CH
        pltpu.sync_copy(idx_hbm.at[pl.ds(rb * TOPK, CH * TOPK)], idx_v)
        pltpu.sync_copy(wx_hbm.at[pl.ds(rb * TOPK, CH * TOPK)], wx_v)
        return pltpu.async_copy(cb_hbm.at[idx_v], rows_v, sem)

    def compute(c, wx_v, rows_v):
        rb = base + c * CH

        def row_body(r, _):
            accs = [jnp.zeros((NL,), jnp.float32) for _ in range(DIM // NL)]
            for k in range(TOPK):
                wv = wx_v[r * TOPK + k]
                for d in range(DIM // NL):
                    accs[d] = accs[d] + wv * rows_v[r * TOPK + k,
                                                    pl.ds(d * NL, NL)]
            for d in range(DIM // NL):
                acc_v[r, pl.ds(d * NL, NL)] = accs[d]
            return ()

        lax.fori_loop(0, CH, row_body, (), unroll=False)
        pltpu.sync_copy(acc_v, out_hbm.at[pl.ds(rb, CH)])

    last = NCHUNK - 1
    fetch(0, idx_a, wx_a, rows_a, sem_a)
    fetch(1, idx_b, wx_b, rows_b, sem_b)

    def superstep(g, _):
        c0 = 2 * g
        pltpu.make_async_copy(cb_hbm.at[idx_a], rows_a, sem_a).wait()
        compute(c0, wx_a, rows_a)
        fetch(jnp.minimum(c0 + 2, last), idx_a, wx_a, rows_a, sem_a)
        pltpu.make_async_copy(cb_hbm.at[idx_b], rows_b, sem_b).wait()
        compute(c0 + 1, wx_b, rows_b)
        fetch(jnp.minimum(c0 + 3, last), idx_b, wx_b, rows_b, sem_b)
        return ()

    lax.fori_loop(0, NSUPER, superstep, (), unroll=False)
    pltpu.make_async_copy(cb_hbm.at[idx_a], rows_a, sem_a).wait()
    pltpu.make_async_copy(cb_hbm.at[idx_b], rows_b, sem_b).wait()


@jax.jit
def kernel(feat_vec, codebook):
    fn = feat_vec / jnp.maximum(
        jnp.linalg.norm(feat_vec, axis=-1, keepdims=True), 1e-12)
    cbn = codebook / jnp.maximum(
        jnp.linalg.norm(codebook, axis=-1, keepdims=True), 1e-12)
    fnb = fn.astype(jnp.bfloat16)
    cbnb = cbn.astype(jnp.bfloat16)

    assign, logits, idx, wx = pl.pallas_call(
        _tc_kernel,
        grid=(B // BR,),
        in_specs=[
            pl.BlockSpec((BR, DIM), lambda i: (i, 0)),
            pl.BlockSpec((NUM_PROTO, DIM), lambda i: (0, 0)),
        ],
        out_specs=[
            pl.BlockSpec((BR, NUM_PROTO), lambda i: (i, 0)),
            pl.BlockSpec((BR, NUM_PROTO), lambda i: (i, 0)),
            pl.BlockSpec((BR, TOPK), lambda i: (i, 0)),
            pl.BlockSpec((BR, TOPK * NL), lambda i: (i, 0)),
        ],
        out_shape=[
            jax.ShapeDtypeStruct((B, NUM_PROTO), jnp.float32),
            jax.ShapeDtypeStruct((B, NUM_PROTO), jnp.float32),
            jax.ShapeDtypeStruct((B, TOPK), jnp.int32),
            jax.ShapeDtypeStruct((B, TOPK * NL), jnp.float32),
        ],
    )(fnb, cbnb)

    mesh = plsc.VectorSubcoreMesh(core_axis_name="c", subcore_axis_name="s")
    proto = pl.kernel(
        _sc_proto,
        out_type=jax.ShapeDtypeStruct((B, DIM), jnp.float32),
        mesh=mesh,
        scratch_types=[
            pltpu.VMEM((CH * TOPK,), jnp.int32),
            pltpu.VMEM((CH * TOPK,), jnp.int32),
            pltpu.VMEM((CH * TOPK, NL), jnp.float32),
            pltpu.VMEM((CH * TOPK, NL), jnp.float32),
            pltpu.VMEM((CH * TOPK, DIM), jnp.float32),
            pltpu.VMEM((CH * TOPK, DIM), jnp.float32),
            pltpu.VMEM((CH, DIM), jnp.float32),
            pltpu.SemaphoreType.DMA,
            pltpu.SemaphoreType.DMA,
        ],
    )(codebook, idx.reshape(B * TOPK), wx.reshape(B * TOPK, NL))
    return (assign, proto, logits)
